# baseline (speedup 1.0000x reference)
import functools

import jax
import jax.numpy as jnp
from jax import lax
from jax.experimental import pallas as pl
from jax.experimental.pallas import tpu as pltpu
from jax.experimental.pallas import tpu_sc as plsc

NC = 2
NS = 16
NW = NC * NS
BLK = 64
CHN = 160


def _seg_sum_sc(src_flat, dst_flat, feat):
    e_wp = src_flat.shape[0] // NW
    n, d = feat.shape
    n_blk = e_wp // BLK
    n_chunk = n_blk // CHN
    c_words = CHN * BLK
    osz = 80
    tile_stride = ((n // NS) // 8) * 8
    tile_span = n - (NS - 1) * tile_stride
    assert tile_span % osz == 0 and tile_stride % 8 == 0
    assert n_blk % CHN == 0 and n_chunk >= 1 and e_wp % 8 == 0

    mesh = plsc.VectorSubcoreMesh(core_axis_name="c", subcore_axis_name="s",
                                  num_cores=NC, num_subcores=NS)

    @functools.partial(
        pl.kernel,
        out_type=jax.ShapeDtypeStruct((NC, n, d), jnp.float32),
        mesh=mesh,
        scratch_types=[
            pltpu.VMEM((c_words,), jnp.int32),
            pltpu.VMEM((c_words,), jnp.int32),
            pltpu.VMEM((c_words,), jnp.int32),
            pltpu.VMEM((c_words,), jnp.int32),
            pltpu.VMEM((BLK,), jnp.int32),
            pltpu.VMEM((BLK,), jnp.int32),
            pltpu.VMEM((BLK, d), jnp.float32),
            pltpu.VMEM((BLK, d), jnp.float32),
            pltpu.VMEM_SHARED((n + 128, d), jnp.float32),
            pltpu.SemaphoreType.DMA,
            pltpu.SemaphoreType.DMA,
            pltpu.SemaphoreType.DMA,
            pltpu.SemaphoreType.DMA,
            pltpu.SemaphoreType.DMA,
        ],
    )
    def seg_kernel(src_hbm, dst_hbm, feat_hbm, out_hbm, src_c0, src_c1,
                   dst_c0, dst_c1, dst_blk0, dst_blk1, rows0, rows1, acc,
                   gsem0, gsem1, isem0, isem1, bsem):
        src_c = [src_c0, src_c1]
        dst_c = [dst_c0, dst_c1]
        dst_blk = [dst_blk0, dst_blk1]
        rows = [rows0, rows1]
        gsem = [gsem0, gsem1]
        isem = [isem0, isem1]
        zbuf = rows[0]
        c = lax.axis_index("c")
        s = lax.axis_index("s")
        wid = c * NS + s
        ibase = wid * e_wp

        def idx_load(k):
            return [
                pltpu.make_async_copy(
                    hbm.at[pl.ds(ibase + k * c_words, c_words)],
                    buf[k % 2], isem[k % 2])
                for hbm, buf in ((src_hbm, src_c), (dst_hbm, dst_c))
            ]

        def start(cps):
            for cp in cps:
                cp.start()

        def wait(cps):
            for cp in cps:
                cp.wait()

        start(idx_load(0))
        if n_chunk > 1:
            start(idx_load(1))

        def zero_body(i, _):
            r = i // (d // 16)
            col = (i % (d // 16)) * 16
            zbuf[r, pl.ds(col, 16)] = jnp.zeros((16,), jnp.float32)
            return 0
        lax.fori_loop(0, osz * (d // 16), zero_body, 0)
        base = s * tile_stride
        zero_copies = [
            pltpu.make_async_copy(zbuf.at[pl.ds(0, osz)],
                                  acc.at[pl.ds(base + k * osz, osz)], bsem)
            for k in range(tile_span // osz)
        ]
        start(zero_copies)
        wait(zero_copies)
        plsc.subcore_barrier()

        def gather_start(off, ip, i):
            pltpu.async_copy(
                feat_hbm.at[src_c[ip].at[pl.ds(off * BLK, BLK)]],
                rows[i], gsem[i])

        def gather_wait(i):
            pltpu.make_async_copy(
                feat_hbm.at[src_c[0].at[pl.ds(0, BLK)]],
                rows[i], gsem[i]).wait()

        def scatter(off, ip, i):
            blk_ref = dst_blk[i]
            for j in range(BLK // 16):
                blk_ref[pl.ds(j * 16, 16)] = dst_c[ip][
                    pl.ds(off * BLK + j * 16, 16)]
            pltpu.sync_copy(rows[i], acc.at[blk_ref], add=True)

        wait(idx_load(0))
        gather_start(0, 0, 0)
        gather_start(1, 0, 1)
        for ch in range(n_chunk):
            ip = ch % 2
            ipn = (ch + 1) % 2

            def pair_body(p, _, ip=ip):
                off = p * 2
                gather_wait(0)
                scatter(off, ip, 0)
                gather_start(off + 2, ip, 0)
                gather_wait(1)
                scatter(off + 1, ip, 1)
                gather_start(off + 3, ip, 1)
                return 0
            lax.fori_loop(0, CHN // 2 - 1, pair_body, 0)

            gather_wait(0)
            scatter(CHN - 2, ip, 0)
            if ch + 1 < n_chunk:
                wait(idx_load(ch + 1))
                gather_start(0, ipn, 0)
            gather_wait(1)
            scatter(CHN - 1, ip, 1)
            if ch + 1 < n_chunk:
                gather_start(1, ipn, 1)
            if ch + 2 < n_chunk:
                start(idx_load(ch + 2))
        plsc.subcore_barrier()

        out_copies = [
            pltpu.make_async_copy(acc.at[pl.ds(base + k * osz, osz)],
                                  out_hbm.at[c, pl.ds(base + k * osz, osz)],
                                  bsem)
            for k in range(tile_span // osz)
        ]
        start(out_copies)
        wait(out_copies)

    return seg_kernel(src_flat, dst_flat, feat)


def _transform_tc(partials, W):
    nc, n, d = partials.shape
    h = W.shape[0]

    def body(p_ref, w_ref, o_ref):
        acc = p_ref[0]
        for i in range(1, nc):
            acc = acc + p_ref[i]
        wm = w_ref[0]
        for i in range(1, h):
            wm = wm + w_ref[i]
        wm = wm * (1.0 / h)
        o_ref[...] = jnp.maximum(
            jax.lax.dot(acc, wm, preferred_element_type=jnp.float32), 0.0)

    return pl.pallas_call(
        body,
        out_shape=jax.ShapeDtypeStruct((n, W.shape[2]), jnp.float32),
    )(partials, W)


def kernel(feat, edge_index, W, al, ar):
    del al, ar
    n, d = feat.shape
    e = edge_index.shape[1]
    assert e % NW == 0 and n % NS == 0 and d % 16 == 0
    e_w = e // NW
    pad = (-e_w) % (BLK * CHN)
    src2 = edge_index[0].reshape(NW, e_w)
    dst2 = edge_index[1].reshape(NW, e_w)
    if pad:
        src2 = jnp.pad(src2, ((0, 0), (0, pad)))
        scrap = n + (jnp.arange(pad, dtype=jnp.int32) % 128)
        dst2 = jnp.concatenate(
            [dst2, jnp.broadcast_to(scrap, (NW, pad))], axis=1)
    partials = _seg_sum_sc(src2.reshape(-1), dst2.reshape(-1), feat)
    return _transform_tc(partials, W)


# speedup vs baseline: 2.9400x; 2.9400x over previous
import functools

import jax
import jax.numpy as jnp
from jax import lax
from jax.experimental import pallas as pl
from jax.experimental.pallas import tpu as pltpu
from jax.experimental.pallas import tpu_sc as plsc

NC = 2
NS = 16
NW = NC * NS
BLK = 80


def _seg_sum_sc(src_flat, dst_flat, feat):
    e = src_flat.shape[0]
    n, d = feat.shape
    e_w = e // NW
    n_blk = e_w // BLK
    tile_stride = ((n // NS) // 8) * 8
    tile_span = n - (NS - 1) * tile_stride
    assert tile_span % BLK == 0 and tile_stride % 8 == 0
    assert n_blk % 2 == 1 and n_blk >= 3 and e_w % 8 == 0

    mesh = plsc.VectorSubcoreMesh(core_axis_name="c", subcore_axis_name="s",
                                  num_cores=NC, num_subcores=NS)

    @functools.partial(
        pl.kernel,
        out_type=jax.ShapeDtypeStruct((NC, n, d), jnp.float32),
        mesh=mesh,
        scratch_types=[
            pltpu.VMEM((e_w,), jnp.int32),
            pltpu.VMEM((e_w,), jnp.int32),
            pltpu.VMEM((BLK,), jnp.int32),
            pltpu.VMEM((BLK,), jnp.int32),
            pltpu.VMEM((BLK, d), jnp.float32),
            pltpu.VMEM((BLK, d), jnp.float32),
            pltpu.VMEM_SHARED((n, d), jnp.float32),
            pltpu.SemaphoreType.DMA,
            pltpu.SemaphoreType.DMA,
            pltpu.SemaphoreType.DMA,
            pltpu.SemaphoreType.DMA,
        ],
    )
    def seg_kernel(src_hbm, dst_hbm, feat_hbm, out_hbm, src_v, dst_v,
                   dst_blk_a, dst_blk_b, rows_a, rows_b, acc,
                   sem_a, sem_b, sem_sa, sem_sb):
        c = lax.axis_index("c")
        s = lax.axis_index("s")
        wid = c * NS + s

        pltpu.async_copy(src_hbm.at[pl.ds(wid * e_w, e_w)], src_v, sem_sa)
        pltpu.async_copy(dst_hbm.at[pl.ds(wid * e_w, e_w)], dst_v, sem_sb)

        def zero_body(i, _):
            r = i // (d // 16)
            col = (i % (d // 16)) * 16
            rows_a[r, pl.ds(col, 16)] = jnp.zeros((16,), jnp.float32)
            return 0
        lax.fori_loop(0, BLK * (d // 16), zero_body, 0)
        base = s * tile_stride
        zero_copies = [
            pltpu.make_async_copy(rows_a, acc.at[pl.ds(base + k * BLK, BLK)],
                                  sem_b)
            for k in range(tile_span // BLK)
        ]
        for cp in zero_copies:
            cp.start()
        for cp in zero_copies:
            cp.wait()
        pltpu.make_async_copy(src_hbm.at[pl.ds(wid * e_w, e_w)], src_v,
                              sem_sa).wait()
        pltpu.make_async_copy(dst_hbm.at[pl.ds(wid * e_w, e_w)], dst_v,
                              sem_sb).wait()
        plsc.subcore_barrier()


        def src_idx(b):
            return src_v.at[pl.ds(b * BLK, BLK)]

        def fill(blk_ref, b):
            for j in range(BLK // 16):
                blk_ref[pl.ds(j * 16, 16)] = dst_v[pl.ds(b * BLK + j * 16,
                                                         16)]

        def wait_gather(rows, sem):
            pltpu.make_async_copy(feat_hbm.at[src_idx(0)], rows, sem).wait()

        def scatter(b, rows, blk_ref):
            fill(blk_ref, b)
            pltpu.sync_copy(rows, acc.at[blk_ref], add=True)

        assert n_blk % 2 == 1 and n_blk >= 3
        pltpu.async_copy(feat_hbm.at[src_idx(0)], rows_a, sem_a)

        def pair_body(g, _):
            b0 = g * 2
            b1 = b0 + 1
            pltpu.async_copy(feat_hbm.at[src_idx(b1)], rows_b, sem_b)
            wait_gather(rows_a, sem_a)
            scatter(b0, rows_a, dst_blk_a)
            pltpu.async_copy(feat_hbm.at[src_idx(b0 + 2)], rows_a, sem_a)
            wait_gather(rows_b, sem_b)
            scatter(b1, rows_b, dst_blk_b)
            return 0
        lax.fori_loop(0, (n_blk - 1) // 2, pair_body, 0)

        wait_gather(rows_a, sem_a)
        scatter(n_blk - 1, rows_a, dst_blk_a)
        plsc.subcore_barrier()

        out_copies = [
            pltpu.make_async_copy(acc.at[pl.ds(base + k * BLK, BLK)],
                                  out_hbm.at[c, pl.ds(base + k * BLK, BLK)],
                                  sem_a)
            for k in range(tile_span // BLK)
        ]
        for cp in out_copies:
            cp.start()
        for cp in out_copies:
            cp.wait()

    return seg_kernel(src_flat, dst_flat, feat)


def _transform_tc(partials, W):
    nc, n, d = partials.shape
    h = W.shape[0]

    def body(p_ref, w_ref, o_ref):
        acc = p_ref[0]
        for i in range(1, nc):
            acc = acc + p_ref[i]
        wm = w_ref[0]
        for i in range(1, h):
            wm = wm + w_ref[i]
        wm = wm * (1.0 / h)
        o_ref[...] = jnp.maximum(
            jax.lax.dot(acc, wm, preferred_element_type=jnp.float32), 0.0)

    return pl.pallas_call(
        body,
        out_shape=jax.ShapeDtypeStruct((n, W.shape[2]), jnp.float32),
    )(partials, W)


def kernel(feat, edge_index, W, al, ar):
    del al, ar
    n, d = feat.shape
    e = edge_index.shape[1]
    assert e % (NW * BLK) == 0 and n % NS == 0 and d % 16 == 0
    partials = _seg_sum_sc(edge_index[0], edge_index[1], feat)
    return _transform_tc(partials, W)


# speedup vs baseline: 2.9409x; 1.0003x over previous
import functools

import jax
import jax.numpy as jnp
from jax import lax
from jax.experimental import pallas as pl
from jax.experimental.pallas import tpu as pltpu
from jax.experimental.pallas import tpu_sc as plsc

NC = 2
NS = 16
NW = NC * NS
BLK = 80


def _seg_sum_sc(src_flat, dst_flat, feat):
    e = src_flat.shape[0]
    n, d = feat.shape
    e_w = e // NW
    n_blk = e_w // BLK
    tile_stride = ((n // NS) // 8) * 8
    tile_span = n - (NS - 1) * tile_stride
    assert tile_span % BLK == 0 and tile_stride % 8 == 0
    assert n_blk % 2 == 1 and n_blk >= 3 and e_w % 8 == 0

    mesh = plsc.VectorSubcoreMesh(core_axis_name="c", subcore_axis_name="s",
                                  num_cores=NC, num_subcores=NS)

    @functools.partial(
        pl.kernel,
        out_type=jax.ShapeDtypeStruct((NC, n, d), jnp.float32),
        mesh=mesh,
        scratch_types=[
            pltpu.VMEM((e_w,), jnp.int32),
            pltpu.VMEM((e_w,), jnp.int32),
            pltpu.VMEM((BLK,), jnp.int32),
            pltpu.VMEM((BLK,), jnp.int32),
            pltpu.VMEM((BLK, d), jnp.float32),
            pltpu.VMEM((BLK, d), jnp.float32),
            pltpu.VMEM_SHARED((n + 128, d), jnp.float32),
            pltpu.SemaphoreType.DMA,
            pltpu.SemaphoreType.DMA,
            pltpu.SemaphoreType.DMA,
            pltpu.SemaphoreType.DMA,
        ],
    )
    def seg_kernel(src_hbm, dst_hbm, feat_hbm, out_hbm, src_v, dst_v,
                   dst_blk_a, dst_blk_b, rows_a, rows_b, acc,
                   sem_a, sem_b, sem_sa, sem_sb):
        c = lax.axis_index("c")
        s = lax.axis_index("s")
        wid = c * NS + s

        pltpu.async_copy(src_hbm.at[pl.ds(wid * e_w, e_w)], src_v, sem_sa)
        pltpu.async_copy(dst_hbm.at[pl.ds(wid * e_w, e_w)], dst_v, sem_sb)

        def zero_body(i, _):
            r = i // (d // 16)
            col = (i % (d // 16)) * 16
            rows_a[r, pl.ds(col, 16)] = jnp.zeros((16,), jnp.float32)
            return 0
        lax.fori_loop(0, BLK * (d // 16), zero_body, 0)
        base = s * tile_stride
        zero_copies = [
            pltpu.make_async_copy(rows_a, acc.at[pl.ds(base + k * BLK, BLK)],
                                  sem_b)
            for k in range(tile_span // BLK)
        ]
        for cp in zero_copies:
            cp.start()
        for cp in zero_copies:
            cp.wait()
        pltpu.make_async_copy(src_hbm.at[pl.ds(wid * e_w, e_w)], src_v,
                              sem_sa).wait()
        pltpu.make_async_copy(dst_hbm.at[pl.ds(wid * e_w, e_w)], dst_v,
                              sem_sb).wait()
        plsc.subcore_barrier()


        def src_idx(b):
            return src_v.at[pl.ds(b * BLK, BLK)]

        def fill(blk_ref, b):
            for j in range(BLK // 16):
                blk_ref[pl.ds(j * 16, 16)] = dst_v[pl.ds(b * BLK + j * 16,
                                                         16)]

        def wait_gather(rows, sem):
            pltpu.make_async_copy(feat_hbm.at[src_idx(0)], rows, sem).wait()

        def scatter(b, rows, blk_ref):
            fill(blk_ref, b)
            pltpu.sync_copy(rows, acc.at[blk_ref], add=True)

        assert n_blk % 2 == 1 and n_blk >= 3
        pltpu.async_copy(feat_hbm.at[src_idx(0)], rows_a, sem_a)

        def pair_body(g, _):
            b0 = g * 2
            b1 = b0 + 1
            pltpu.async_copy(feat_hbm.at[src_idx(b1)], rows_b, sem_b)
            wait_gather(rows_a, sem_a)
            scatter(b0, rows_a, dst_blk_a)
            pltpu.async_copy(feat_hbm.at[src_idx(b0 + 2)], rows_a, sem_a)
            wait_gather(rows_b, sem_b)
            scatter(b1, rows_b, dst_blk_b)
            return 0
        lax.fori_loop(0, (n_blk - 1) // 2, pair_body, 0)

        wait_gather(rows_a, sem_a)
        scatter(n_blk - 1, rows_a, dst_blk_a)
        plsc.subcore_barrier()

        out_copies = [
            pltpu.make_async_copy(acc.at[pl.ds(base + k * BLK, BLK)],
                                  out_hbm.at[c, pl.ds(base + k * BLK, BLK)],
                                  sem_a)
            for k in range(tile_span // BLK)
        ]
        for cp in out_copies:
            cp.start()
        for cp in out_copies:
            cp.wait()

    return seg_kernel(src_flat, dst_flat, feat)


def _transform_tc(partials, W):
    nc, n, d = partials.shape
    h = W.shape[0]

    def body(p_ref, w_ref, o_ref):
        acc = p_ref[0]
        for i in range(1, nc):
            acc = acc + p_ref[i]
        wm = w_ref[0]
        for i in range(1, h):
            wm = wm + w_ref[i]
        wm = wm * (1.0 / h)
        o_ref[...] = jnp.maximum(
            jax.lax.dot(acc, wm, preferred_element_type=jnp.float32), 0.0)

    return pl.pallas_call(
        body,
        out_shape=jax.ShapeDtypeStruct((n, W.shape[2]), jnp.float32),
    )(partials, W)


def kernel(feat, edge_index, W, al, ar):
    del al, ar
    n, d = feat.shape
    e = edge_index.shape[1]
    assert e % (NW * BLK) == 0 and n % NS == 0 and d % 16 == 0
    partials = _seg_sum_sc(edge_index[0], edge_index[1], feat)
    return _transform_tc(partials, W)
